# baseline (speedup 1.0000x reference)
import jax
import jax.numpy as jnp
from jax.experimental import pallas as pl

_N = 512
_U = 128


def _prep_body(adj_ref, adjt_ref, x_ref, poi_ref, wpoi_ref, bp_ref,
               s_ref, x1_ref, x2_ref, pb_ref):
    adj = adj_ref[...]
    s_ref[...] = adj / jnp.clip(jnp.sum(adj, axis=1, keepdims=True), 1e-8, None)
    adjt = adjt_ref[...]
    st = adjt / jnp.clip(jnp.sum(adjt, axis=0, keepdims=True), 1e-8, None)
    x0 = x_ref[...]
    x1 = jnp.dot(x0, st)
    x2 = 2.0 * jnp.dot(x1, st) - x0
    x1_ref[...] = x1
    x2_ref[...] = x2
    pb_ref[...] = jnp.dot(poi_ref[...], wpoi_ref[...]) + bp_ref[0, 0]


def _dot(a, b):
    return jnp.dot(a.astype(jnp.bfloat16), b.astype(jnp.bfloat16),
                   preferred_element_type=jnp.float32)


def _main_body(s_ref, x0_ref, x1_ref, x2_ref,
               h0_ref, h1_ref,
               wg0h_ref, wg0x_ref, bg0_ref,
               wc0h_ref, wc0x_ref, bc0_ref,
               wg1_ref, bg1_ref, wc1_ref, bc1_ref,
               wph_ref, pb_ref,
               out_ref, h0o_ref, h1o_ref):
    s = s_ref[...].astype(jnp.bfloat16)
    h0 = h0_ref[0]
    h1 = h1_ref[0]
    x0 = x0_ref[0]
    x1 = x1_ref[0]
    x2 = x2_ref[0]

    g1 = _dot(s, h0)
    g2 = 2.0 * _dot(s, g1) - h0
    gate = _dot(jnp.concatenate([h0, g1, g2], axis=1), wg0h_ref[...])
    gate = gate + x0 * wg0x_ref[0:1] + x1 * wg0x_ref[1:2] + x2 * wg0x_ref[2:3]
    gate = jax.nn.sigmoid(gate + bg0_ref[...])
    r = gate[:, :_U]
    u = gate[:, _U:]
    rh = r * h0
    c1 = _dot(s, rh)
    c2 = 2.0 * _dot(s, c1) - rh
    cand = _dot(jnp.concatenate([rh, c1, c2], axis=1), wc0h_ref[...])
    cand = cand + x0 * wc0x_ref[0:1] + x1 * wc0x_ref[1:2] + x2 * wc0x_ref[2:3]
    cand = jnp.tanh(cand + bc0_ref[...])
    hn0 = u * h0 + (1.0 - u) * cand

    xx0 = jnp.concatenate([hn0, h1], axis=1)
    xx1 = _dot(s, xx0)
    xx2 = 2.0 * _dot(s, xx1) - xx0
    gate1 = _dot(jnp.concatenate([xx0, xx1, xx2], axis=1), wg1_ref[...])
    gate1 = jax.nn.sigmoid(gate1 + bg1_ref[...])
    r1 = gate1[:, :_U]
    u1 = gate1[:, _U:]
    rh1 = r1 * h1
    y1 = _dot(s, rh1)
    y2 = 2.0 * _dot(s, y1) - rh1
    yc = jnp.concatenate([hn0, rh1, xx1[:, :_U], y1, xx2[:, :_U], y2], axis=1)
    cand1 = jnp.tanh(_dot(yc, wc1_ref[...]) + bc1_ref[...])
    hn1 = u1 * h1 + (1.0 - u1) * cand1

    out_ref[0] = _dot(hn1, wph_ref[...]) + pb_ref[...]
    h0o_ref[0] = hn0
    h1o_ref[0] = hn1


def kernel(inputs, adj_mx, nodevec1, nodevec2, POI_feat, labels,
           hidden_state, W_gate0, b_gate0, W_cand0, b_cand0,
           W_gate1, b_gate1, W_cand1, b_cand1, W_proj, b_proj):
    B = inputs.shape[0]
    f32 = jnp.float32

    s, x1t, x2t, pb = pl.pallas_call(
        _prep_body,
        out_shape=[
            jax.ShapeDtypeStruct((_N, _N), f32),
            jax.ShapeDtypeStruct((B, _N), f32),
            jax.ShapeDtypeStruct((B, _N), f32),
            jax.ShapeDtypeStruct((_N, 1), f32),
        ],
    )(adj_mx, adj_mx.T, inputs, POI_feat, W_proj[_U:], b_proj.reshape(1, 1))

    wg0h = jnp.concatenate([W_gate0[1:129], W_gate0[130:258], W_gate0[259:387]], axis=0)
    wg0x = jnp.stack([W_gate0[0], W_gate0[129], W_gate0[258]])
    wc0h = jnp.concatenate([W_cand0[1:129], W_cand0[130:258], W_cand0[259:387]], axis=0)
    wc0x = jnp.stack([W_cand0[0], W_cand0[129], W_cand0[258]])

    const2 = lambda shape: pl.BlockSpec(shape, lambda b: (0, 0))
    step3 = lambda shape: pl.BlockSpec(shape, lambda b: (b, 0, 0))

    out, h0o, h1o = pl.pallas_call(
        _main_body,
        grid=(B,),
        in_specs=[
            const2((_N, _N)),
            step3((1, _N, 1)), step3((1, _N, 1)), step3((1, _N, 1)),
            step3((1, _N, _U)), step3((1, _N, _U)),
            const2((3 * _U, 2 * _U)), const2((3, 2 * _U)), const2((1, 2 * _U)),
            const2((3 * _U, _U)), const2((3, _U)), const2((1, _U)),
            const2((6 * _U, 2 * _U)), const2((1, 2 * _U)),
            const2((6 * _U, _U)), const2((1, _U)),
            const2((_U, 1)), const2((_N, 1)),
        ],
        out_specs=[
            step3((1, _N, 1)),
            step3((1, _N, _U)), step3((1, _N, _U)),
        ],
        out_shape=[
            jax.ShapeDtypeStruct((B, _N, 1), f32),
            jax.ShapeDtypeStruct((B, _N, _U), f32),
            jax.ShapeDtypeStruct((B, _N, _U), f32),
        ],
    )(
        s,
        inputs.reshape(B, _N, 1), x1t.reshape(B, _N, 1), x2t.reshape(B, _N, 1),
        hidden_state[0].reshape(B, _N, _U), hidden_state[1].reshape(B, _N, _U),
        wg0h, wg0x, b_gate0.reshape(1, 2 * _U),
        wc0h, wc0x, b_cand0.reshape(1, _U),
        W_gate1, b_gate1.reshape(1, 2 * _U),
        W_cand1, b_cand1.reshape(1, _U),
        W_proj[:_U], pb,
    )

    out_final = out.reshape(B, _N)
    hidden = jnp.stack([h0o.reshape(B, _N * _U), h1o.reshape(B, _N * _U)])
    return (out_final, hidden)


# speedup vs baseline: 1.1693x; 1.1693x over previous
import jax
import jax.numpy as jnp
from jax.experimental import pallas as pl

_N = 512
_U = 128


def _prep_body(adj_ref, adjt_ref, x_ref, poi_ref, wpoi_ref, bp_ref,
               s_ref, x1_ref, x2_ref, pb_ref):
    adj = adj_ref[...]
    s_ref[...] = adj / jnp.clip(jnp.sum(adj, axis=1, keepdims=True), 1e-8, None)
    adjt = adjt_ref[...]
    st = adjt / jnp.clip(jnp.sum(adjt, axis=0, keepdims=True), 1e-8, None)
    x0 = x_ref[...]
    x1 = jnp.dot(x0, st)
    x2 = 2.0 * jnp.dot(x1, st) - x0
    x1_ref[...] = x1
    x2_ref[...] = x2
    pb_ref[...] = jnp.dot(poi_ref[...], wpoi_ref[...]) + bp_ref[0, 0]


def _dot(a, b):
    return jnp.dot(a.astype(jnp.bfloat16), b.astype(jnp.bfloat16),
                   preferred_element_type=jnp.float32)


_BT = 2


def _main_body(s_ref, x0_ref, x1_ref, x2_ref,
               h0_ref, h1_ref,
               wg0h_ref, wg0x_ref, bg0_ref,
               wc0h_ref, wc0x_ref, bc0_ref,
               wg1_ref, bg1_ref, wc1_ref, bc1_ref,
               wph_ref, pb_ref,
               out_ref, h0o_ref, h1o_ref):
    s = s_ref[...].astype(jnp.bfloat16)
    h0l = [h0_ref[i] for i in range(_BT)]
    h1l = [h1_ref[i] for i in range(_BT)]
    x0s = jnp.concatenate([x0_ref[i] for i in range(_BT)], axis=0)
    x1s = jnp.concatenate([x1_ref[i] for i in range(_BT)], axis=0)
    x2s = jnp.concatenate([x2_ref[i] for i in range(_BT)], axis=0)

    h0c = jnp.concatenate(h0l, axis=1)
    h0s = jnp.concatenate(h0l, axis=0)
    g1c = _dot(s, h0c)
    g2c = 2.0 * _dot(s, g1c) - h0c
    xch = jnp.concatenate(
        [jnp.concatenate([h0c[:, i*_U:(i+1)*_U], g1c[:, i*_U:(i+1)*_U],
                          g2c[:, i*_U:(i+1)*_U]], axis=1)
         for i in range(_BT)], axis=0)
    gate = _dot(xch, wg0h_ref[...])
    gate = gate + x0s * wg0x_ref[0:1] + x1s * wg0x_ref[1:2] + x2s * wg0x_ref[2:3]
    gate = jax.nn.sigmoid(gate + bg0_ref[...])
    r = gate[:, :_U]
    u = gate[:, _U:]
    rh = r * h0s
    rhc = jnp.concatenate([rh[i*_N:(i+1)*_N] for i in range(_BT)], axis=1)
    c1c = _dot(s, rhc)
    c2c = 2.0 * _dot(s, c1c) - rhc
    xcc = jnp.concatenate(
        [jnp.concatenate([rhc[:, i*_U:(i+1)*_U], c1c[:, i*_U:(i+1)*_U],
                          c2c[:, i*_U:(i+1)*_U]], axis=1)
         for i in range(_BT)], axis=0)
    cand = _dot(xcc, wc0h_ref[...])
    cand = cand + x0s * wc0x_ref[0:1] + x1s * wc0x_ref[1:2] + x2s * wc0x_ref[2:3]
    cand = jnp.tanh(cand + bc0_ref[...])
    hn0 = u * h0s + (1.0 - u) * cand

    hn0l = [hn0[i*_N:(i+1)*_N] for i in range(_BT)]
    xx0c = jnp.concatenate(
        [jnp.concatenate([hn0l[i], h1l[i]], axis=1) for i in range(_BT)],
        axis=1)
    xx1c = _dot(s, xx0c)
    xx2c = 2.0 * _dot(s, xx1c) - xx0c
    w = 2 * _U
    xc1 = jnp.concatenate(
        [jnp.concatenate([xx0c[:, i*w:(i+1)*w], xx1c[:, i*w:(i+1)*w],
                          xx2c[:, i*w:(i+1)*w]], axis=1)
         for i in range(_BT)], axis=0)
    gate1 = jax.nn.sigmoid(_dot(xc1, wg1_ref[...]) + bg1_ref[...])
    r1 = gate1[:, :_U]
    u1 = gate1[:, _U:]
    h1s = jnp.concatenate(h1l, axis=0)
    rh1 = r1 * h1s
    rh1c = jnp.concatenate([rh1[i*_N:(i+1)*_N] for i in range(_BT)], axis=1)
    y1c = _dot(s, rh1c)
    y2c = 2.0 * _dot(s, y1c) - rh1c
    yc = jnp.concatenate(
        [jnp.concatenate([hn0l[i], rh1c[:, i*_U:(i+1)*_U],
                          xx1c[:, i*w:i*w+_U], y1c[:, i*_U:(i+1)*_U],
                          xx2c[:, i*w:i*w+_U], y2c[:, i*_U:(i+1)*_U]], axis=1)
         for i in range(_BT)], axis=0)
    cand1 = jnp.tanh(_dot(yc, wc1_ref[...]) + bc1_ref[...])
    hn1 = u1 * h1s + (1.0 - u1) * cand1

    proj = _dot(hn1, wph_ref[...])
    for i in range(_BT):
        out_ref[i] = proj[i*_N:(i+1)*_N] + pb_ref[...]
        h0o_ref[i] = hn0[i*_N:(i+1)*_N]
        h1o_ref[i] = hn1[i*_N:(i+1)*_N]


def kernel(inputs, adj_mx, nodevec1, nodevec2, POI_feat, labels,
           hidden_state, W_gate0, b_gate0, W_cand0, b_cand0,
           W_gate1, b_gate1, W_cand1, b_cand1, W_proj, b_proj):
    B = inputs.shape[0]
    f32 = jnp.float32

    s, x1t, x2t, pb = pl.pallas_call(
        _prep_body,
        out_shape=[
            jax.ShapeDtypeStruct((_N, _N), f32),
            jax.ShapeDtypeStruct((B, _N), f32),
            jax.ShapeDtypeStruct((B, _N), f32),
            jax.ShapeDtypeStruct((_N, 1), f32),
        ],
    )(adj_mx, adj_mx.T, inputs, POI_feat, W_proj[_U:], b_proj.reshape(1, 1))

    wg0h = jnp.concatenate([W_gate0[1:129], W_gate0[130:258], W_gate0[259:387]], axis=0)
    wg0x = jnp.stack([W_gate0[0], W_gate0[129], W_gate0[258]])
    wc0h = jnp.concatenate([W_cand0[1:129], W_cand0[130:258], W_cand0[259:387]], axis=0)
    wc0x = jnp.stack([W_cand0[0], W_cand0[129], W_cand0[258]])

    const2 = lambda shape: pl.BlockSpec(shape, lambda b: (0, 0))
    step3 = lambda shape: pl.BlockSpec(shape, lambda b: (b, 0, 0))

    out, h0o, h1o = pl.pallas_call(
        _main_body,
        grid=(B // _BT,),
        in_specs=[
            const2((_N, _N)),
            step3((_BT, _N, 1)), step3((_BT, _N, 1)), step3((_BT, _N, 1)),
            step3((_BT, _N, _U)), step3((_BT, _N, _U)),
            const2((3 * _U, 2 * _U)), const2((3, 2 * _U)), const2((1, 2 * _U)),
            const2((3 * _U, _U)), const2((3, _U)), const2((1, _U)),
            const2((6 * _U, 2 * _U)), const2((1, 2 * _U)),
            const2((6 * _U, _U)), const2((1, _U)),
            const2((_U, 1)), const2((_N, 1)),
        ],
        out_specs=[
            step3((_BT, _N, 1)),
            step3((_BT, _N, _U)), step3((_BT, _N, _U)),
        ],
        out_shape=[
            jax.ShapeDtypeStruct((B, _N, 1), f32),
            jax.ShapeDtypeStruct((B, _N, _U), f32),
            jax.ShapeDtypeStruct((B, _N, _U), f32),
        ],
    )(
        s,
        inputs.reshape(B, _N, 1), x1t.reshape(B, _N, 1), x2t.reshape(B, _N, 1),
        hidden_state[0].reshape(B, _N, _U), hidden_state[1].reshape(B, _N, _U),
        wg0h, wg0x, b_gate0.reshape(1, 2 * _U),
        wc0h, wc0x, b_cand0.reshape(1, _U),
        W_gate1, b_gate1.reshape(1, 2 * _U),
        W_cand1, b_cand1.reshape(1, _U),
        W_proj[:_U], pb,
    )

    out_final = out.reshape(B, _N)
    hidden = jnp.stack([h0o.reshape(B, _N * _U), h1o.reshape(B, _N * _U)])
    return (out_final, hidden)


# speedup vs baseline: 1.2833x; 1.0975x over previous
import jax
import jax.numpy as jnp
from jax.experimental import pallas as pl

_N = 512
_U = 128


def _prep_body(adj_ref, adjt_ref, x_ref, poi_ref, wpoi_ref, bp_ref,
               s_ref, x1_ref, x2_ref, pb_ref):
    adj = adj_ref[...]
    s_ref[...] = adj / jnp.clip(jnp.sum(adj, axis=1, keepdims=True), 1e-8, None)
    adjt = adjt_ref[...]
    st = adjt / jnp.clip(jnp.sum(adjt, axis=0, keepdims=True), 1e-8, None)
    x0 = x_ref[...]
    x1 = jnp.dot(x0, st)
    x2 = 2.0 * jnp.dot(x1, st) - x0
    x1_ref[...] = x1
    x2_ref[...] = x2
    pb_ref[...] = jnp.dot(poi_ref[...], wpoi_ref[...]) + bp_ref[0, 0]


def _dot(a, b):
    return jnp.dot(a.astype(jnp.bfloat16), b.astype(jnp.bfloat16),
                   preferred_element_type=jnp.float32)


_BT = 4


def _main_body(s_ref, x0_ref, x1_ref, x2_ref,
               h0_ref, h1_ref,
               wg0h_ref, wg0x_ref, bg0_ref,
               wc0h_ref, wc0x_ref, bc0_ref,
               wg1_ref, bg1_ref, wc1_ref, bc1_ref,
               wph_ref, pb_ref,
               out_ref, h0o_ref, h1o_ref):
    s = s_ref[...].astype(jnp.bfloat16)
    h0l = [h0_ref[i] for i in range(_BT)]
    h1l = [h1_ref[i] for i in range(_BT)]
    x0s = jnp.concatenate([x0_ref[i] for i in range(_BT)], axis=0)
    x1s = jnp.concatenate([x1_ref[i] for i in range(_BT)], axis=0)
    x2s = jnp.concatenate([x2_ref[i] for i in range(_BT)], axis=0)

    h0c = jnp.concatenate(h0l, axis=1)
    h0s = jnp.concatenate(h0l, axis=0)
    g1c = _dot(s, h0c)
    g2c = 2.0 * _dot(s, g1c) - h0c
    xch = jnp.concatenate(
        [jnp.concatenate([h0c[:, i*_U:(i+1)*_U], g1c[:, i*_U:(i+1)*_U],
                          g2c[:, i*_U:(i+1)*_U]], axis=1)
         for i in range(_BT)], axis=0)
    gate = _dot(xch, wg0h_ref[...])
    gate = gate + x0s * wg0x_ref[0:1] + x1s * wg0x_ref[1:2] + x2s * wg0x_ref[2:3]
    gate = jax.nn.sigmoid(gate + bg0_ref[...])
    r = gate[:, :_U]
    u = gate[:, _U:]
    rh = r * h0s
    rhc = jnp.concatenate([rh[i*_N:(i+1)*_N] for i in range(_BT)], axis=1)
    c1c = _dot(s, rhc)
    c2c = 2.0 * _dot(s, c1c) - rhc
    xcc = jnp.concatenate(
        [jnp.concatenate([rhc[:, i*_U:(i+1)*_U], c1c[:, i*_U:(i+1)*_U],
                          c2c[:, i*_U:(i+1)*_U]], axis=1)
         for i in range(_BT)], axis=0)
    cand = _dot(xcc, wc0h_ref[...])
    cand = cand + x0s * wc0x_ref[0:1] + x1s * wc0x_ref[1:2] + x2s * wc0x_ref[2:3]
    cand = jnp.tanh(cand + bc0_ref[...])
    hn0 = u * h0s + (1.0 - u) * cand

    hn0l = [hn0[i*_N:(i+1)*_N] for i in range(_BT)]
    xx0c = jnp.concatenate(
        [jnp.concatenate([hn0l[i], h1l[i]], axis=1) for i in range(_BT)],
        axis=1)
    xx1c = _dot(s, xx0c)
    xx2c = 2.0 * _dot(s, xx1c) - xx0c
    w = 2 * _U
    xc1 = jnp.concatenate(
        [jnp.concatenate([xx0c[:, i*w:(i+1)*w], xx1c[:, i*w:(i+1)*w],
                          xx2c[:, i*w:(i+1)*w]], axis=1)
         for i in range(_BT)], axis=0)
    gate1 = jax.nn.sigmoid(_dot(xc1, wg1_ref[...]) + bg1_ref[...])
    r1 = gate1[:, :_U]
    u1 = gate1[:, _U:]
    h1s = jnp.concatenate(h1l, axis=0)
    rh1 = r1 * h1s
    rh1c = jnp.concatenate([rh1[i*_N:(i+1)*_N] for i in range(_BT)], axis=1)
    y1c = _dot(s, rh1c)
    y2c = 2.0 * _dot(s, y1c) - rh1c
    yc = jnp.concatenate(
        [jnp.concatenate([hn0l[i], rh1c[:, i*_U:(i+1)*_U],
                          xx1c[:, i*w:i*w+_U], y1c[:, i*_U:(i+1)*_U],
                          xx2c[:, i*w:i*w+_U], y2c[:, i*_U:(i+1)*_U]], axis=1)
         for i in range(_BT)], axis=0)
    cand1 = jnp.tanh(_dot(yc, wc1_ref[...]) + bc1_ref[...])
    hn1 = u1 * h1s + (1.0 - u1) * cand1

    proj = _dot(hn1, wph_ref[...])
    for i in range(_BT):
        out_ref[i] = proj[i*_N:(i+1)*_N] + pb_ref[...]
        h0o_ref[i] = hn0[i*_N:(i+1)*_N]
        h1o_ref[i] = hn1[i*_N:(i+1)*_N]


def kernel(inputs, adj_mx, nodevec1, nodevec2, POI_feat, labels,
           hidden_state, W_gate0, b_gate0, W_cand0, b_cand0,
           W_gate1, b_gate1, W_cand1, b_cand1, W_proj, b_proj):
    B = inputs.shape[0]
    f32 = jnp.float32

    s, x1t, x2t, pb = pl.pallas_call(
        _prep_body,
        out_shape=[
            jax.ShapeDtypeStruct((_N, _N), f32),
            jax.ShapeDtypeStruct((B, _N), f32),
            jax.ShapeDtypeStruct((B, _N), f32),
            jax.ShapeDtypeStruct((_N, 1), f32),
        ],
    )(adj_mx, adj_mx.T, inputs, POI_feat, W_proj[_U:], b_proj.reshape(1, 1))

    wg0h = jnp.concatenate([W_gate0[1:129], W_gate0[130:258], W_gate0[259:387]], axis=0)
    wg0x = jnp.stack([W_gate0[0], W_gate0[129], W_gate0[258]])
    wc0h = jnp.concatenate([W_cand0[1:129], W_cand0[130:258], W_cand0[259:387]], axis=0)
    wc0x = jnp.stack([W_cand0[0], W_cand0[129], W_cand0[258]])

    const2 = lambda shape: pl.BlockSpec(shape, lambda b: (0, 0))
    step3 = lambda shape: pl.BlockSpec(shape, lambda b: (b, 0, 0))

    out, h0o, h1o = pl.pallas_call(
        _main_body,
        grid=(B // _BT,),
        in_specs=[
            const2((_N, _N)),
            step3((_BT, _N, 1)), step3((_BT, _N, 1)), step3((_BT, _N, 1)),
            step3((_BT, _N, _U)), step3((_BT, _N, _U)),
            const2((3 * _U, 2 * _U)), const2((3, 2 * _U)), const2((1, 2 * _U)),
            const2((3 * _U, _U)), const2((3, _U)), const2((1, _U)),
            const2((6 * _U, 2 * _U)), const2((1, 2 * _U)),
            const2((6 * _U, _U)), const2((1, _U)),
            const2((_U, 1)), const2((_N, 1)),
        ],
        out_specs=[
            step3((_BT, _N, 1)),
            step3((_BT, _N, _U)), step3((_BT, _N, _U)),
        ],
        out_shape=[
            jax.ShapeDtypeStruct((B, _N, 1), f32),
            jax.ShapeDtypeStruct((B, _N, _U), f32),
            jax.ShapeDtypeStruct((B, _N, _U), f32),
        ],
    )(
        s,
        inputs.reshape(B, _N, 1), x1t.reshape(B, _N, 1), x2t.reshape(B, _N, 1),
        hidden_state[0].reshape(B, _N, _U), hidden_state[1].reshape(B, _N, _U),
        wg0h, wg0x, b_gate0.reshape(1, 2 * _U),
        wc0h, wc0x, b_cand0.reshape(1, _U),
        W_gate1, b_gate1.reshape(1, 2 * _U),
        W_cand1, b_cand1.reshape(1, _U),
        W_proj[:_U], pb,
    )

    out_final = out.reshape(B, _N)
    hidden = jnp.stack([h0o.reshape(B, _N * _U), h1o.reshape(B, _N * _U)])
    return (out_final, hidden)


# speedup vs baseline: 1.3369x; 1.0418x over previous
import jax
import jax.numpy as jnp
from jax.experimental import pallas as pl

_N = 512
_U = 128


def _prep_body(adj_ref, adjt_ref, x_ref, poi_ref, wpoi_ref, bp_ref,
               s_ref, x1_ref, x2_ref, pb_ref):
    adj = adj_ref[...]
    s_ref[...] = adj / jnp.clip(jnp.sum(adj, axis=1, keepdims=True), 1e-8, None)
    adjt = adjt_ref[...]
    st = adjt / jnp.clip(jnp.sum(adjt, axis=0, keepdims=True), 1e-8, None)
    x0 = x_ref[...]
    x1 = jnp.dot(x0, st)
    x2 = 2.0 * jnp.dot(x1, st) - x0
    x1_ref[...] = x1
    x2_ref[...] = x2
    pb_ref[...] = jnp.dot(poi_ref[...], wpoi_ref[...]) + bp_ref[0, 0]


def _dot(a, b):
    return jnp.dot(a.astype(jnp.bfloat16), b.astype(jnp.bfloat16),
                   preferred_element_type=jnp.float32)


_BT = 8


def _main_body(s_ref, x0_ref, x1_ref, x2_ref,
               h0_ref, h1_ref,
               wg0h_ref, wg0x_ref, bg0_ref,
               wc0h_ref, wc0x_ref, bc0_ref,
               wg1_ref, bg1_ref, wc1_ref, bc1_ref,
               wph_ref, pb_ref,
               out_ref, h0o_ref, h1o_ref):
    s = s_ref[...].astype(jnp.bfloat16)
    h0l = [h0_ref[i] for i in range(_BT)]
    h1l = [h1_ref[i] for i in range(_BT)]
    x0s = jnp.concatenate([x0_ref[i] for i in range(_BT)], axis=0)
    x1s = jnp.concatenate([x1_ref[i] for i in range(_BT)], axis=0)
    x2s = jnp.concatenate([x2_ref[i] for i in range(_BT)], axis=0)

    h0c = jnp.concatenate(h0l, axis=1)
    h0s = jnp.concatenate(h0l, axis=0)
    g1c = _dot(s, h0c)
    g2c = 2.0 * _dot(s, g1c) - h0c
    xch = jnp.concatenate(
        [jnp.concatenate([h0c[:, i*_U:(i+1)*_U], g1c[:, i*_U:(i+1)*_U],
                          g2c[:, i*_U:(i+1)*_U]], axis=1)
         for i in range(_BT)], axis=0)
    gate = _dot(xch, wg0h_ref[...])
    gate = gate + x0s * wg0x_ref[0:1] + x1s * wg0x_ref[1:2] + x2s * wg0x_ref[2:3]
    gate = jax.nn.sigmoid(gate + bg0_ref[...])
    r = gate[:, :_U]
    u = gate[:, _U:]
    rh = r * h0s
    rhc = jnp.concatenate([rh[i*_N:(i+1)*_N] for i in range(_BT)], axis=1)
    c1c = _dot(s, rhc)
    c2c = 2.0 * _dot(s, c1c) - rhc
    xcc = jnp.concatenate(
        [jnp.concatenate([rhc[:, i*_U:(i+1)*_U], c1c[:, i*_U:(i+1)*_U],
                          c2c[:, i*_U:(i+1)*_U]], axis=1)
         for i in range(_BT)], axis=0)
    cand = _dot(xcc, wc0h_ref[...])
    cand = cand + x0s * wc0x_ref[0:1] + x1s * wc0x_ref[1:2] + x2s * wc0x_ref[2:3]
    cand = jnp.tanh(cand + bc0_ref[...])
    hn0 = u * h0s + (1.0 - u) * cand

    hn0l = [hn0[i*_N:(i+1)*_N] for i in range(_BT)]
    xx0c = jnp.concatenate(
        [jnp.concatenate([hn0l[i], h1l[i]], axis=1) for i in range(_BT)],
        axis=1)
    xx1c = _dot(s, xx0c)
    xx2c = 2.0 * _dot(s, xx1c) - xx0c
    w = 2 * _U
    xc1 = jnp.concatenate(
        [jnp.concatenate([xx0c[:, i*w:(i+1)*w], xx1c[:, i*w:(i+1)*w],
                          xx2c[:, i*w:(i+1)*w]], axis=1)
         for i in range(_BT)], axis=0)
    gate1 = jax.nn.sigmoid(_dot(xc1, wg1_ref[...]) + bg1_ref[...])
    r1 = gate1[:, :_U]
    u1 = gate1[:, _U:]
    h1s = jnp.concatenate(h1l, axis=0)
    rh1 = r1 * h1s
    rh1c = jnp.concatenate([rh1[i*_N:(i+1)*_N] for i in range(_BT)], axis=1)
    y1c = _dot(s, rh1c)
    y2c = 2.0 * _dot(s, y1c) - rh1c
    yc = jnp.concatenate(
        [jnp.concatenate([hn0l[i], rh1c[:, i*_U:(i+1)*_U],
                          xx1c[:, i*w:i*w+_U], y1c[:, i*_U:(i+1)*_U],
                          xx2c[:, i*w:i*w+_U], y2c[:, i*_U:(i+1)*_U]], axis=1)
         for i in range(_BT)], axis=0)
    cand1 = jnp.tanh(_dot(yc, wc1_ref[...]) + bc1_ref[...])
    hn1 = u1 * h1s + (1.0 - u1) * cand1

    proj = _dot(hn1, wph_ref[...])
    for i in range(_BT):
        out_ref[i] = proj[i*_N:(i+1)*_N] + pb_ref[...]
        h0o_ref[i] = hn0[i*_N:(i+1)*_N]
        h1o_ref[i] = hn1[i*_N:(i+1)*_N]


def kernel(inputs, adj_mx, nodevec1, nodevec2, POI_feat, labels,
           hidden_state, W_gate0, b_gate0, W_cand0, b_cand0,
           W_gate1, b_gate1, W_cand1, b_cand1, W_proj, b_proj):
    B = inputs.shape[0]
    f32 = jnp.float32

    s, x1t, x2t, pb = pl.pallas_call(
        _prep_body,
        out_shape=[
            jax.ShapeDtypeStruct((_N, _N), f32),
            jax.ShapeDtypeStruct((B, _N), f32),
            jax.ShapeDtypeStruct((B, _N), f32),
            jax.ShapeDtypeStruct((_N, 1), f32),
        ],
    )(adj_mx, adj_mx.T, inputs, POI_feat, W_proj[_U:], b_proj.reshape(1, 1))

    wg0h = jnp.concatenate([W_gate0[1:129], W_gate0[130:258], W_gate0[259:387]], axis=0)
    wg0x = jnp.stack([W_gate0[0], W_gate0[129], W_gate0[258]])
    wc0h = jnp.concatenate([W_cand0[1:129], W_cand0[130:258], W_cand0[259:387]], axis=0)
    wc0x = jnp.stack([W_cand0[0], W_cand0[129], W_cand0[258]])

    const2 = lambda shape: pl.BlockSpec(shape, lambda b: (0, 0))
    step3 = lambda shape: pl.BlockSpec(shape, lambda b: (b, 0, 0))

    out, h0o, h1o = pl.pallas_call(
        _main_body,
        grid=(B // _BT,),
        in_specs=[
            const2((_N, _N)),
            step3((_BT, _N, 1)), step3((_BT, _N, 1)), step3((_BT, _N, 1)),
            step3((_BT, _N, _U)), step3((_BT, _N, _U)),
            const2((3 * _U, 2 * _U)), const2((3, 2 * _U)), const2((1, 2 * _U)),
            const2((3 * _U, _U)), const2((3, _U)), const2((1, _U)),
            const2((6 * _U, 2 * _U)), const2((1, 2 * _U)),
            const2((6 * _U, _U)), const2((1, _U)),
            const2((_U, 1)), const2((_N, 1)),
        ],
        out_specs=[
            step3((_BT, _N, 1)),
            step3((_BT, _N, _U)), step3((_BT, _N, _U)),
        ],
        out_shape=[
            jax.ShapeDtypeStruct((B, _N, 1), f32),
            jax.ShapeDtypeStruct((B, _N, _U), f32),
            jax.ShapeDtypeStruct((B, _N, _U), f32),
        ],
    )(
        s,
        inputs.reshape(B, _N, 1), x1t.reshape(B, _N, 1), x2t.reshape(B, _N, 1),
        hidden_state[0].reshape(B, _N, _U), hidden_state[1].reshape(B, _N, _U),
        wg0h, wg0x, b_gate0.reshape(1, 2 * _U),
        wc0h, wc0x, b_cand0.reshape(1, _U),
        W_gate1, b_gate1.reshape(1, 2 * _U),
        W_cand1, b_cand1.reshape(1, _U),
        W_proj[:_U], pb,
    )

    out_final = out.reshape(B, _N)
    hidden = jnp.stack([h0o.reshape(B, _N * _U), h1o.reshape(B, _N * _U)])
    return (out_final, hidden)


# speedup vs baseline: 1.4414x; 1.0782x over previous
import jax
import jax.numpy as jnp
from jax.experimental import pallas as pl

_N = 512
_U = 128


def _prep_body(adj_ref, adjt_ref, x_ref, poi_ref, wpoi_ref, bp_ref,
               s_ref, x1_ref, x2_ref, pb_ref):
    adj = adj_ref[...]
    s = adj / jnp.clip(jnp.sum(adj, axis=1, keepdims=True), 1e-8, None)
    s_ref[...] = s.astype(jnp.bfloat16)
    adjt = adjt_ref[...]
    st = adjt / jnp.clip(jnp.sum(adjt, axis=0, keepdims=True), 1e-8, None)
    x0 = x_ref[...]
    x1 = jnp.dot(x0, st)
    x2 = 2.0 * jnp.dot(x1, st) - x0
    x1_ref[...] = x1
    x2_ref[...] = x2
    pb_ref[...] = jnp.dot(poi_ref[...], wpoi_ref[...]) + bp_ref[0, 0]


def _dot(a, b):
    return jnp.dot(a.astype(jnp.bfloat16), b.astype(jnp.bfloat16),
                   preferred_element_type=jnp.float32)


_BT = 8


def _main_body(s_ref, x0_ref, x1_ref, x2_ref,
               h0_ref, h1_ref,
               wg0h_ref, wg0x_ref, bg0_ref,
               wc0h_ref, wc0x_ref, bc0_ref,
               wg1_ref, bg1_ref, wc1_ref, bc1_ref,
               wph_ref, pb_ref,
               out_ref, ho_ref):
    s = s_ref[...]
    h0l = [h0_ref[i] for i in range(_BT)]
    h1l = [h1_ref[i] for i in range(_BT)]
    x0s = jnp.concatenate([x0_ref[i] for i in range(_BT)], axis=0)
    x1s = jnp.concatenate([x1_ref[i] for i in range(_BT)], axis=0)
    x2s = jnp.concatenate([x2_ref[i] for i in range(_BT)], axis=0)

    h0c = jnp.concatenate(h0l, axis=1)
    h0s = jnp.concatenate(h0l, axis=0)
    g1c = _dot(s, h0c)
    g2c = 2.0 * _dot(s, g1c) - h0c
    xch = jnp.concatenate(
        [jnp.concatenate([h0c[:, i*_U:(i+1)*_U], g1c[:, i*_U:(i+1)*_U],
                          g2c[:, i*_U:(i+1)*_U]], axis=1)
         for i in range(_BT)], axis=0)
    gate = _dot(xch, wg0h_ref[...])
    gate = gate + x0s * wg0x_ref[0:1] + x1s * wg0x_ref[1:2] + x2s * wg0x_ref[2:3]
    gate = jax.nn.sigmoid(gate + bg0_ref[...])
    r = gate[:, :_U]
    u = gate[:, _U:]
    rh = r * h0s
    rhc = jnp.concatenate([rh[i*_N:(i+1)*_N] for i in range(_BT)], axis=1)
    c1c = _dot(s, rhc)
    c2c = 2.0 * _dot(s, c1c) - rhc
    xcc = jnp.concatenate(
        [jnp.concatenate([rhc[:, i*_U:(i+1)*_U], c1c[:, i*_U:(i+1)*_U],
                          c2c[:, i*_U:(i+1)*_U]], axis=1)
         for i in range(_BT)], axis=0)
    cand = _dot(xcc, wc0h_ref[...])
    cand = cand + x0s * wc0x_ref[0:1] + x1s * wc0x_ref[1:2] + x2s * wc0x_ref[2:3]
    cand = jnp.tanh(cand + bc0_ref[...])
    hn0 = u * h0s + (1.0 - u) * cand

    hn0l = [hn0[i*_N:(i+1)*_N] for i in range(_BT)]
    xx0c = jnp.concatenate(
        [jnp.concatenate([hn0l[i], h1l[i]], axis=1) for i in range(_BT)],
        axis=1)
    xx1c = _dot(s, xx0c)
    xx2c = 2.0 * _dot(s, xx1c) - xx0c
    w = 2 * _U
    xc1 = jnp.concatenate(
        [jnp.concatenate([xx0c[:, i*w:(i+1)*w], xx1c[:, i*w:(i+1)*w],
                          xx2c[:, i*w:(i+1)*w]], axis=1)
         for i in range(_BT)], axis=0)
    gate1 = jax.nn.sigmoid(_dot(xc1, wg1_ref[...]) + bg1_ref[...])
    r1 = gate1[:, :_U]
    u1 = gate1[:, _U:]
    h1s = jnp.concatenate(h1l, axis=0)
    rh1 = r1 * h1s
    rh1c = jnp.concatenate([rh1[i*_N:(i+1)*_N] for i in range(_BT)], axis=1)
    y1c = _dot(s, rh1c)
    y2c = 2.0 * _dot(s, y1c) - rh1c
    yc = jnp.concatenate(
        [jnp.concatenate([hn0l[i], rh1c[:, i*_U:(i+1)*_U],
                          xx1c[:, i*w:i*w+_U], y1c[:, i*_U:(i+1)*_U],
                          xx2c[:, i*w:i*w+_U], y2c[:, i*_U:(i+1)*_U]], axis=1)
         for i in range(_BT)], axis=0)
    cand1 = jnp.tanh(_dot(yc, wc1_ref[...]) + bc1_ref[...])
    hn1 = u1 * h1s + (1.0 - u1) * cand1

    proj = _dot(hn1, wph_ref[...])
    for i in range(_BT):
        out_ref[i] = proj[i*_N:(i+1)*_N] + pb_ref[...]
        ho_ref[0, i] = hn0[i*_N:(i+1)*_N]
        ho_ref[1, i] = hn1[i*_N:(i+1)*_N]


def kernel(inputs, adj_mx, nodevec1, nodevec2, POI_feat, labels,
           hidden_state, W_gate0, b_gate0, W_cand0, b_cand0,
           W_gate1, b_gate1, W_cand1, b_cand1, W_proj, b_proj):
    B = inputs.shape[0]
    f32 = jnp.float32

    s, x1t, x2t, pb = pl.pallas_call(
        _prep_body,
        out_shape=[
            jax.ShapeDtypeStruct((_N, _N), jnp.bfloat16),
            jax.ShapeDtypeStruct((B, _N), f32),
            jax.ShapeDtypeStruct((B, _N), f32),
            jax.ShapeDtypeStruct((_N, 1), f32),
        ],
    )(adj_mx, adj_mx.T, inputs, POI_feat, W_proj[_U:], b_proj.reshape(1, 1))

    wg0h = jnp.concatenate([W_gate0[1:129], W_gate0[130:258], W_gate0[259:387]], axis=0)
    wg0x = jnp.stack([W_gate0[0], W_gate0[129], W_gate0[258]])
    wc0h = jnp.concatenate([W_cand0[1:129], W_cand0[130:258], W_cand0[259:387]], axis=0)
    wc0x = jnp.stack([W_cand0[0], W_cand0[129], W_cand0[258]])

    const2 = lambda shape: pl.BlockSpec(shape, lambda b: (0, 0))
    step3 = lambda shape: pl.BlockSpec(shape, lambda b: (b, 0, 0))

    out, ho = pl.pallas_call(
        _main_body,
        grid=(B // _BT,),
        in_specs=[
            const2((_N, _N)),
            step3((_BT, _N, 1)), step3((_BT, _N, 1)), step3((_BT, _N, 1)),
            step3((_BT, _N, _U)), step3((_BT, _N, _U)),
            const2((3 * _U, 2 * _U)), const2((3, 2 * _U)), const2((1, 2 * _U)),
            const2((3 * _U, _U)), const2((3, _U)), const2((1, _U)),
            const2((6 * _U, 2 * _U)), const2((1, 2 * _U)),
            const2((6 * _U, _U)), const2((1, _U)),
            const2((_U, 1)), const2((_N, 1)),
        ],
        out_specs=[
            step3((_BT, _N, 1)),
            pl.BlockSpec((2, _BT, _N, _U), lambda b: (0, b, 0, 0)),
        ],
        out_shape=[
            jax.ShapeDtypeStruct((B, _N, 1), f32),
            jax.ShapeDtypeStruct((2, B, _N, _U), f32),
        ],
    )(
        s,
        inputs.reshape(B, _N, 1), x1t.reshape(B, _N, 1), x2t.reshape(B, _N, 1),
        hidden_state[0].reshape(B, _N, _U), hidden_state[1].reshape(B, _N, _U),
        wg0h.astype(jnp.bfloat16), wg0x, b_gate0.reshape(1, 2 * _U),
        wc0h.astype(jnp.bfloat16), wc0x, b_cand0.reshape(1, _U),
        W_gate1.astype(jnp.bfloat16), b_gate1.reshape(1, 2 * _U),
        W_cand1.astype(jnp.bfloat16), b_cand1.reshape(1, _U),
        W_proj[:_U].astype(jnp.bfloat16), pb,
    )

    out_final = out.reshape(B, _N)
    hidden = ho.reshape(2, B, _N * _U)
    return (out_final, hidden)
